# baseline (speedup 1.0000x reference)
import functools

import jax
import jax.numpy as jnp
from jax import lax
from jax.experimental import pallas as pl
from jax.experimental.pallas import tpu as pltpu
from jax.experimental.pallas import tpu_sc as plsc

_B = 16384
_F = 26
_V = 100000
_D = 32
_TOT = _F * _D
_R = _B * _F
_NC, _NS = 2, 16
_NW = _NC * _NS

_VB = 4096


_KPF = 25600


def _tp_body(x_ref, o_ref):
    x = x_ref[...][0]
    eye = jnp.eye(32, dtype=jnp.float32)
    acc = None
    for r in range(4):
        sel = jnp.pad(eye, ((0, 0), (r * 32, 96 - r * 32)))
        part = jnp.dot(jnp.transpose(x[:, r * 1024:(r + 1) * 1024], (1, 0)),
                       sel, preferred_element_type=jnp.float32)
        acc = part if acc is None else acc + part
    o_ref[...] = acc[None]


def _transpose_tbl(tabT):
    return pl.pallas_call(
        _tp_body,
        grid=(_F, (_V + _VB - 1) // _VB),
        in_specs=[pl.BlockSpec((1, _D, _VB), lambda f, v: (f, 0, v))],
        out_specs=pl.BlockSpec((1, _VB // 4, 128), lambda f, v: (f, v, 0)),
        out_shape=jax.ShapeDtypeStruct((_F, _KPF, 128), jnp.float32),
    )(tabT)


_ROWS_W = _R // _NW
_CHUNK = 128
_NCH = _ROWS_W // _CHUNK
_NBUF = 4


def _gather_body(kidx, didx, tbl, out, klist, dofs, staged, outv, gsem):
    w = lax.axis_index("s") * _NC + lax.axis_index("c")
    base = w * _ROWS_W
    pltpu.sync_copy(kidx.at[pl.ds(base, _ROWS_W)],
                    klist.at[pl.ds(0, _ROWS_W)])
    pltpu.sync_copy(didx.at[pl.ds(base, _ROWS_W)],
                    dofs.at[pl.ds(0, _ROWS_W)])

    def extract(b, j):
        def row(i, carry):
            doff = dofs[pl.ds(j * _CHUNK + i, 16)][0]
            co = (i & 3) * 32
            outv[i >> 2, pl.ds(co, 16)] = staged[b, i, pl.ds(doff, 16)]
            outv[i >> 2, pl.ds(co + 16, 16)] = (
                staged[b, i, pl.ds(doff + 16, 16)])
            return carry
        lax.fori_loop(0, _CHUNK, row, 0)

    def fire(j, b):
        pltpu.async_copy(tbl.at[klist.at[pl.ds(j * _CHUNK, _CHUNK)]],
                         staged.at[b], gsem)

    def proc(j, b):
        pltpu.make_async_copy(tbl.at[klist.at[pl.ds(0, _CHUNK)]],
                              staged.at[b], gsem).wait()
        extract(b, j)
        pltpu.sync_copy(outv, out.at[pl.ds(w * 3328 + j * 32, 32)])

    for b in range(_NBUF):
        fire(b, b)

    def grpf(g, carry):
        for b in range(_NBUF):
            j = g * _NBUF + b
            proc(j, b)
            fire(j + _NBUF, b)
        return carry

    lax.fori_loop(0, _NCH // _NBUF - 1, grpf, 0)
    for b in range(_NBUF):
        proc(_NCH - _NBUF + b, b)


def _gather(kidx, didx, tbl128):
    mesh = plsc.VectorSubcoreMesh(core_axis_name="c", subcore_axis_name="s")
    return pl.kernel(
        _gather_body,
        mesh=mesh,
        out_type=jax.ShapeDtypeStruct((_R * _D // 128, 128), jnp.float32),
        scratch_types=[
            pltpu.VMEM((_ROWS_W + 16,), jnp.int32),
            pltpu.VMEM((_ROWS_W + 16,), jnp.int32),
            pltpu.VMEM((_NBUF, _CHUNK, 128), jnp.float32),
            pltpu.VMEM((32, 128), jnp.float32),
            pltpu.SemaphoreType.DMA,
        ],
    )(kidx, didx, tbl128)


_BLK = 2048


def _mlp_body(x_ref, w1_ref, b1_ref, w2_ref, b2_ref, w3_ref, b3_ref, o_ref):
    x = x_ref[...]
    h = jnp.maximum(
        jnp.dot(x, w1_ref[...], preferred_element_type=jnp.float32)
        + b1_ref[...], 0.0)
    h = jnp.maximum(
        jnp.dot(h, w2_ref[...], preferred_element_type=jnp.float32)
        + b2_ref[...], 0.0)
    o_ref[...] = (jnp.dot(h, w3_ref[...], preferred_element_type=jnp.float32)
                  + b3_ref[...])


def _mlp(x, W1, b1, W2, b2, W3, b3):
    return pl.pallas_call(
        _mlp_body,
        grid=(_B // _BLK,),
        in_specs=[
            pl.BlockSpec((_BLK, _TOT), lambda i: (i, 0)),
            pl.BlockSpec((_TOT, 32), lambda i: (0, 0)),
            pl.BlockSpec((1, 32), lambda i: (0, 0)),
            pl.BlockSpec((32, 16), lambda i: (0, 0)),
            pl.BlockSpec((1, 16), lambda i: (0, 0)),
            pl.BlockSpec((16, 10), lambda i: (0, 0)),
            pl.BlockSpec((1, 10), lambda i: (0, 0)),
        ],
        out_specs=pl.BlockSpec((_BLK, 10), lambda i: (i, 0)),
        out_shape=jax.ShapeDtypeStruct((_B, 10), jnp.float32),
    )(x, W1, b1.reshape(1, 32), W2, b2.reshape(1, 16),
      W3, b3.reshape(1, 10))


def kernel(x_cat, tables, W1, b1, W2, b2, W3, b3):
    tabT = jnp.transpose(tables, (0, 2, 1))
    tbl128 = _transpose_tbl(tabT).reshape(_F * _KPF, 128)

    v = x_cat.astype(jnp.int32)
    foffs = (jnp.arange(_F, dtype=jnp.int32) * _KPF)[None, :]
    kidx = (foffs + ((v >> 12) << 10) + (v & 1023)).reshape(_R)
    didx = (((v >> 10) & 3) << 5).reshape(_R)
    x128 = _gather(kidx, didx, tbl128)
    x = x128.reshape(_B, _TOT)
    return _mlp(x, W1, b1, W2, b2, W3, b3)


# speedup vs baseline: 1.2842x; 1.2842x over previous
import functools

import jax
import jax.numpy as jnp
from jax import lax
from jax.experimental import pallas as pl
from jax.experimental.pallas import tpu as pltpu
from jax.experimental.pallas import tpu_sc as plsc

_B = 16384
_F = 26
_V = 100000
_D = 32
_TOT = _F * _D
_R = _B * _F
_NC, _NS = 2, 16
_NW = _NC * _NS

_VB = 8192


_KPF = _V


def _tp_body(x_ref, o_ref):
    x = x_ref[...][0]
    xt = jnp.transpose(x, (1, 0))
    o_ref[...] = jnp.concatenate(
        [xt, jnp.zeros((_VB, 96), jnp.float32)], axis=1)[None]


def _transpose_tbl(tabT):
    return pl.pallas_call(
        _tp_body,
        grid=(_F, (_V + _VB - 1) // _VB),
        in_specs=[pl.BlockSpec((1, _D, _VB), lambda f, v: (f, 0, v))],
        out_specs=pl.BlockSpec((1, _VB, 128), lambda f, v: (f, v, 0)),
        out_shape=jax.ShapeDtypeStruct((_F, _V, 128), jnp.float32),
        compiler_params=pltpu.CompilerParams(
            dimension_semantics=("parallel", "parallel")),
    )(tabT)


_ROWS_W = _R // _NW
_CHUNK = 128
_NCH = _ROWS_W // _CHUNK
_NBUF = 4


def _gather_body(kidx, tbl, out, klist, staged, outv, gsem):
    w = lax.axis_index("s") * _NC + lax.axis_index("c")
    base = w * _ROWS_W
    pltpu.sync_copy(kidx.at[pl.ds(base, _ROWS_W)], klist)

    def extract(b):
        def row(i, carry):
            co = (i & 3) * 32
            outv[i >> 2, pl.ds(co, 16)] = staged[b, i, pl.ds(0, 16)]
            outv[i >> 2, pl.ds(co + 16, 16)] = staged[b, i, pl.ds(16, 16)]
            return carry
        lax.fori_loop(0, _CHUNK, row, 0)

    def fire(j, b):
        pltpu.async_copy(tbl.at[klist.at[pl.ds(j * _CHUNK, _CHUNK)]],
                         staged.at[b], gsem)

    def proc(j, b):
        pltpu.make_async_copy(tbl.at[klist.at[pl.ds(0, _CHUNK)]],
                              staged.at[b], gsem).wait()
        extract(b)
        pltpu.sync_copy(outv, out.at[pl.ds(w * 3328 + j * 32, 32)])

    for b in range(_NBUF):
        fire(b, b)

    def grpf(g, carry):
        for b in range(_NBUF):
            j = g * _NBUF + b
            proc(j, b)
            fire(j + _NBUF, b)
        return carry

    lax.fori_loop(0, _NCH // _NBUF - 1, grpf, 0)
    for b in range(_NBUF):
        proc(_NCH - _NBUF + b, b)


def _gather(kidx, tbl128):
    mesh = plsc.VectorSubcoreMesh(core_axis_name="c", subcore_axis_name="s")
    return pl.kernel(
        _gather_body,
        mesh=mesh,
        out_type=jax.ShapeDtypeStruct((_R * _D // 128, 128), jnp.float32),
        scratch_types=[
            pltpu.VMEM((_ROWS_W,), jnp.int32),
            pltpu.VMEM((_NBUF, _CHUNK, 128), jnp.float32),
            pltpu.VMEM((32, 128), jnp.float32),
            pltpu.SemaphoreType.DMA,
        ],
    )(kidx, tbl128)


_BLK = 2048


def _mlp_body(x_ref, w1_ref, b1_ref, w2_ref, b2_ref, w3_ref, b3_ref, o_ref):
    x = x_ref[...]
    h = jnp.maximum(
        jnp.dot(x, w1_ref[...], preferred_element_type=jnp.float32)
        + b1_ref[...], 0.0)
    h = jnp.maximum(
        jnp.dot(h, w2_ref[...], preferred_element_type=jnp.float32)
        + b2_ref[...], 0.0)
    o_ref[...] = (jnp.dot(h, w3_ref[...], preferred_element_type=jnp.float32)
                  + b3_ref[...])


def _mlp(x, W1, b1, W2, b2, W3, b3):
    return pl.pallas_call(
        _mlp_body,
        grid=(_B // _BLK,),
        in_specs=[
            pl.BlockSpec((_BLK, _TOT), lambda i: (i, 0)),
            pl.BlockSpec((_TOT, 32), lambda i: (0, 0)),
            pl.BlockSpec((1, 32), lambda i: (0, 0)),
            pl.BlockSpec((32, 16), lambda i: (0, 0)),
            pl.BlockSpec((1, 16), lambda i: (0, 0)),
            pl.BlockSpec((16, 10), lambda i: (0, 0)),
            pl.BlockSpec((1, 10), lambda i: (0, 0)),
        ],
        out_specs=pl.BlockSpec((_BLK, 10), lambda i: (i, 0)),
        out_shape=jax.ShapeDtypeStruct((_B, 10), jnp.float32),
    )(x, W1, b1.reshape(1, 32), W2, b2.reshape(1, 16),
      W3, b3.reshape(1, 10))


def kernel(x_cat, tables, W1, b1, W2, b2, W3, b3):
    tabT = jnp.transpose(tables, (0, 2, 1))
    tbl128 = _transpose_tbl(tabT).reshape(_F * _V, 128)

    offs = (jnp.arange(_F, dtype=jnp.int32) * _V)[None, :]
    kidx = (x_cat.astype(jnp.int32) + offs).reshape(_R)
    x128 = _gather(kidx, tbl128)
    x = x128.reshape(_B, _TOT)
    return _mlp(x, W1, b1, W2, b2, W3, b3)


# speedup vs baseline: 1.3947x; 1.0860x over previous
import functools

import jax
import jax.numpy as jnp
from jax import lax
from jax.experimental import pallas as pl
from jax.experimental.pallas import tpu as pltpu
from jax.experimental.pallas import tpu_sc as plsc

_B = 16384
_F = 26
_V = 100000
_D = 32
_TOT = _F * _D
_R = _B * _F
_NC, _NS = 2, 16
_NW = _NC * _NS

_VB = 16384


_KPF = _V


def _tp_body(x_ref, o_ref):
    x = x_ref[...][0]
    xt = jnp.transpose(x, (1, 0))
    o_ref[...] = jnp.concatenate(
        [xt, jnp.zeros((_VB, 96), jnp.float32)], axis=1)[None]


def _transpose_tbl(tabT):
    return pl.pallas_call(
        _tp_body,
        grid=(_F, (_V + _VB - 1) // _VB),
        in_specs=[pl.BlockSpec((1, _D, _VB), lambda f, v: (f, 0, v))],
        out_specs=pl.BlockSpec((1, _VB, 128), lambda f, v: (f, v, 0)),
        out_shape=jax.ShapeDtypeStruct((_F, _V, 128), jnp.float32),
        compiler_params=pltpu.CompilerParams(
            dimension_semantics=("parallel", "parallel")),
    )(tabT)


_ROWS_W = _R // _NW
_CHUNK = 128
_NCH = _ROWS_W // _CHUNK
_NBUF = 4


def _gather_body(kidx, tbl, out, klist, staged, outv, gsem):
    w = lax.axis_index("s") * _NC + lax.axis_index("c")
    base = w * _ROWS_W
    pltpu.sync_copy(kidx.at[pl.ds(base, _ROWS_W)], klist)

    def extract(b):
        def row(i, carry):
            co = (i & 3) * 32
            outv[i >> 2, pl.ds(co, 16)] = staged[b, i, pl.ds(0, 16)]
            outv[i >> 2, pl.ds(co + 16, 16)] = staged[b, i, pl.ds(16, 16)]
            return carry
        lax.fori_loop(0, _CHUNK, row, 0)

    def fire(j, b):
        pltpu.async_copy(tbl.at[klist.at[pl.ds(j * _CHUNK, _CHUNK)]],
                         staged.at[b], gsem)

    def proc(j, b):
        pltpu.make_async_copy(tbl.at[klist.at[pl.ds(0, _CHUNK)]],
                              staged.at[b], gsem).wait()
        extract(b)
        pltpu.sync_copy(outv, out.at[pl.ds(w * 3328 + j * 32, 32)])

    for b in range(_NBUF):
        fire(b, b)

    def grpf(g, carry):
        for b in range(_NBUF):
            j = g * _NBUF + b
            proc(j, b)
            fire(j + _NBUF, b)
        return carry

    lax.fori_loop(0, _NCH // _NBUF - 1, grpf, 0)
    for b in range(_NBUF):
        proc(_NCH - _NBUF + b, b)


def _gather(kidx, tbl128):
    mesh = plsc.VectorSubcoreMesh(core_axis_name="c", subcore_axis_name="s")
    return pl.kernel(
        _gather_body,
        mesh=mesh,
        out_type=jax.ShapeDtypeStruct((_R * _D // 128, 128), jnp.float32),
        scratch_types=[
            pltpu.VMEM((_ROWS_W,), jnp.int32),
            pltpu.VMEM((_NBUF, _CHUNK, 128), jnp.float32),
            pltpu.VMEM((32, 128), jnp.float32),
            pltpu.SemaphoreType.DMA,
        ],
    )(kidx, tbl128)


_BLK = 2048


def _mlp_body(x_ref, w1_ref, b1_ref, w2_ref, b2_ref, w3_ref, b3_ref, o_ref):
    x = x_ref[...]
    h = jnp.maximum(
        jnp.dot(x, w1_ref[...], preferred_element_type=jnp.float32)
        + b1_ref[...], 0.0)
    h = jnp.maximum(
        jnp.dot(h, w2_ref[...], preferred_element_type=jnp.float32)
        + b2_ref[...], 0.0)
    o_ref[...] = (jnp.dot(h, w3_ref[...], preferred_element_type=jnp.float32)
                  + b3_ref[...])


def _mlp(x, W1, b1, W2, b2, W3, b3):
    return pl.pallas_call(
        _mlp_body,
        grid=(_B // _BLK,),
        in_specs=[
            pl.BlockSpec((_BLK, _TOT), lambda i: (i, 0)),
            pl.BlockSpec((_TOT, 32), lambda i: (0, 0)),
            pl.BlockSpec((1, 32), lambda i: (0, 0)),
            pl.BlockSpec((32, 16), lambda i: (0, 0)),
            pl.BlockSpec((1, 16), lambda i: (0, 0)),
            pl.BlockSpec((16, 10), lambda i: (0, 0)),
            pl.BlockSpec((1, 10), lambda i: (0, 0)),
        ],
        out_specs=pl.BlockSpec((_BLK, 10), lambda i: (i, 0)),
        out_shape=jax.ShapeDtypeStruct((_B, 10), jnp.float32),
    )(x, W1, b1.reshape(1, 32), W2, b2.reshape(1, 16),
      W3, b3.reshape(1, 10))


def kernel(x_cat, tables, W1, b1, W2, b2, W3, b3):
    tabT = jnp.transpose(tables, (0, 2, 1))
    tbl128 = _transpose_tbl(tabT).reshape(_F * _V, 128)

    offs = (jnp.arange(_F, dtype=jnp.int32) * _V)[None, :]
    kidx = (x_cat.astype(jnp.int32) + offs).reshape(_R)
    x128 = _gather(kidx, tbl128)
    x = x128.reshape(_B, _TOT)
    return _mlp(x, W1, b1, W2, b2, W3, b3)


# speedup vs baseline: 1.4368x; 1.0302x over previous
import functools

import jax
import jax.numpy as jnp
from jax import lax
from jax.experimental import pallas as pl
from jax.experimental.pallas import tpu as pltpu
from jax.experimental.pallas import tpu_sc as plsc

_B = 16384
_F = 26
_V = 100000
_D = 32
_TOT = _F * _D
_R = _B * _F
_NC, _NS = 2, 16
_NW = _NC * _NS

_VB = 32768


_KPF = _V


def _tp_body(x_ref, o_ref):
    x = x_ref[...][0]
    xt = jnp.transpose(x, (1, 0))
    o_ref[...] = jnp.concatenate(
        [xt, jnp.zeros((_VB, 96), jnp.float32)], axis=1)[None]


def _transpose_tbl(tabT):
    return pl.pallas_call(
        _tp_body,
        grid=(_F, (_V + _VB - 1) // _VB),
        in_specs=[pl.BlockSpec((1, _D, _VB), lambda f, v: (f, 0, v))],
        out_specs=pl.BlockSpec((1, _VB, 128), lambda f, v: (f, v, 0)),
        out_shape=jax.ShapeDtypeStruct((_F, _V, 128), jnp.float32),
        compiler_params=pltpu.CompilerParams(
            dimension_semantics=("parallel", "parallel")),
    )(tabT)


_ROWS_W = _R // _NW
_CHUNK = 104
_NCH = _ROWS_W // _CHUNK
_NBUF = 4


def _gather_body(kidx, tbl, out, klist, staged, outv, gsem):
    w = lax.axis_index("s") * _NC + lax.axis_index("c")
    base = w * _ROWS_W
    pltpu.sync_copy(kidx.at[pl.ds(base, _ROWS_W)], klist)

    def extract(b):
        def row(i, carry):
            ro = i // 52
            co = (i - ro * 52) * 32
            outv[ro, pl.ds(co, 16)] = staged[b, i, pl.ds(0, 16)]
            outv[ro, pl.ds(co + 16, 16)] = staged[b, i, pl.ds(16, 16)]
            return carry
        lax.fori_loop(0, _CHUNK, row, 0)

    def fire(j, b):
        pltpu.async_copy(tbl.at[klist.at[pl.ds(j * _CHUNK, _CHUNK)]],
                         staged.at[b], gsem)

    def proc(j, b):
        pltpu.make_async_copy(tbl.at[klist.at[pl.ds(0, _CHUNK)]],
                              staged.at[b], gsem).wait()
        extract(b)
        pltpu.sync_copy(outv, out.at[pl.ds(w * 256 + j * 2, 2)])

    for b in range(_NBUF):
        fire(b, b)

    def grpf(g, carry):
        for b in range(_NBUF):
            j = g * _NBUF + b
            proc(j, b)
            fire(j + _NBUF, b)
        return carry

    lax.fori_loop(0, _NCH // _NBUF - 1, grpf, 0)
    for b in range(_NBUF):
        proc(_NCH - _NBUF + b, b)


def _gather(kidx, tbl128):
    mesh = plsc.VectorSubcoreMesh(core_axis_name="c", subcore_axis_name="s")
    return pl.kernel(
        _gather_body,
        mesh=mesh,
        out_type=jax.ShapeDtypeStruct((_B // 2, 2 * _TOT), jnp.float32),
        scratch_types=[
            pltpu.VMEM((_ROWS_W,), jnp.int32),
            pltpu.VMEM((_NBUF, _CHUNK, 128), jnp.float32),
            pltpu.VMEM((2, 2 * _TOT), jnp.float32),
            pltpu.SemaphoreType.DMA,
        ],
    )(kidx, tbl128)


_PBLK = 1024


def _mlp_body(x_ref, w1_ref, b1_ref, w2_ref, b2_ref, w3_ref, b3_ref, o_ref):
    x = x_ref[...]
    h = jnp.maximum(
        jnp.dot(x, w1_ref[...], preferred_element_type=jnp.float32)
        + b1_ref[...], 0.0)
    h = jnp.maximum(
        jnp.dot(h, w2_ref[...], preferred_element_type=jnp.float32)
        + b2_ref[...], 0.0)
    o_ref[...] = (jnp.dot(h, w3_ref[...], preferred_element_type=jnp.float32)
                  + b3_ref[...])


def _mlp(x2, W1s, b1s, W2s, b2s, W3s, b3s):
    return pl.pallas_call(
        _mlp_body,
        grid=(_B // 2 // _PBLK,),
        in_specs=[
            pl.BlockSpec((_PBLK, 2 * _TOT), lambda i: (i, 0)),
            pl.BlockSpec((2 * _TOT, 64), lambda i: (0, 0)),
            pl.BlockSpec((1, 64), lambda i: (0, 0)),
            pl.BlockSpec((64, 32), lambda i: (0, 0)),
            pl.BlockSpec((1, 32), lambda i: (0, 0)),
            pl.BlockSpec((32, 20), lambda i: (0, 0)),
            pl.BlockSpec((1, 20), lambda i: (0, 0)),
        ],
        out_specs=pl.BlockSpec((_PBLK, 20), lambda i: (i, 0)),
        out_shape=jax.ShapeDtypeStruct((_B // 2, 20), jnp.float32),
    )(x2, W1s, b1s, W2s, b2s, W3s, b3s)


def _blockdiag2(W):
    n, m = W.shape
    Z = jnp.zeros((n, m), W.dtype)
    return jnp.concatenate(
        [jnp.concatenate([W, Z], axis=1),
         jnp.concatenate([Z, W], axis=1)], axis=0)


def kernel(x_cat, tables, W1, b1, W2, b2, W3, b3):
    tabT = jnp.transpose(tables, (0, 2, 1))
    tbl128 = _transpose_tbl(tabT).reshape(_F * _V, 128)

    offs = (jnp.arange(_F, dtype=jnp.int32) * _V)[None, :]
    kidx = (x_cat.astype(jnp.int32) + offs).reshape(_R)
    x2 = _gather(kidx, tbl128)

    W1s = _blockdiag2(W1)
    W2s = _blockdiag2(W2)
    W3s = _blockdiag2(W3)
    b1s = jnp.concatenate([b1, b1]).reshape(1, 64)
    b2s = jnp.concatenate([b2, b2]).reshape(1, 32)
    b3s = jnp.concatenate([b3, b3]).reshape(1, 20)
    o2 = _mlp(x2, W1s, b1s, W2s, b2s, W3s, b3s)
    return o2.reshape(_B, 10)
